# Initial kernel scaffold; baseline (speedup 1.0000x reference)
import jax
import jax.numpy as jnp
from jax.experimental import pallas as pl


def kernel(x, embedding):
    raise NotImplementedError("write your pallas kernel here")


# speedup vs baseline: 2.8981x; 2.8981x over previous
import jax
import jax.numpy as jnp
from jax import lax
from jax.experimental import pallas as pl
from jax.experimental.pallas import tpu as pltpu
from jax.experimental.pallas import tpu_sc as plsc

NC = 2
NS = 16
NW = NC * NS

CHUNK = 128
D = 128


def _build(B):
    n_per_w = B // NW
    n_chunks = n_per_w // CHUNK

    mesh = plsc.VectorSubcoreMesh(core_axis_name="c", subcore_axis_name="s")

    def body(table_hbm, idx_hbm, out_hbm, idx_v, rows_v, gsem):
        wid = lax.axis_index("s") * NC + lax.axis_index("c")
        pltpu.sync_copy(idx_hbm.at[wid], idx_v)
        pltpu.async_copy(table_hbm.at[idx_v.at[0]], rows_v.at[0], gsem)

        row0 = wid * n_per_w

        def outer(j0, carry):
            for b in range(2):
                j = j0 * 2 + b

                @pl.when(j + 1 < n_chunks)
                def _():
                    pltpu.async_copy(
                        table_hbm.at[idx_v.at[j + 1]], rows_v.at[1 - b], gsem
                    )

                pltpu.make_async_copy(
                    table_hbm.at[idx_v.at[j]], rows_v.at[b], gsem
                ).wait()
                pltpu.sync_copy(
                    rows_v.at[b], out_hbm.at[pl.ds(row0 + j * CHUNK, CHUNK)]
                )
            return carry

        lax.fori_loop(0, n_chunks // 2, outer, 0)

    grid_kernel = pl.kernel(
        body,
        out_type=jax.ShapeDtypeStruct((B, D), jnp.float32),
        mesh=mesh,
        scratch_types=[
            pltpu.VMEM((n_chunks, CHUNK), jnp.int32),
            pltpu.VMEM((2, CHUNK, D), jnp.float32),
            pltpu.SemaphoreType.DMA,
        ],
    )
    return grid_kernel


def kernel(x, embedding):
    orig_shape = x.shape
    flat = x.reshape(-1)
    B = flat.shape[0]
    idx3d = flat.reshape(NW, B // (NW * CHUNK), CHUNK)
    out = _build(B)(embedding, idx3d)
    return out.reshape(*orig_shape, D)


# speedup vs baseline: 2.9059x; 1.0027x over previous
import jax
import jax.numpy as jnp
from jax import lax
from jax.experimental import pallas as pl
from jax.experimental.pallas import tpu as pltpu
from jax.experimental.pallas import tpu_sc as plsc

NC = 2
NS = 16
NW = NC * NS

CHUNK = 128
D = 128


NB = 5


def _build(B):
    n_per_w = B // NW
    n_chunks = n_per_w // CHUNK
    assert n_chunks % NB == 0

    mesh = plsc.VectorSubcoreMesh(core_axis_name="c", subcore_axis_name="s")

    def body(table_hbm, idx_hbm, out_hbm, idx_v, rows_v, gsems, osems):
        wid = lax.axis_index("s") * NC + lax.axis_index("c")
        pltpu.sync_copy(idx_hbm.at[wid], idx_v)
        for p in range(NB - 1):
            pltpu.async_copy(table_hbm.at[idx_v.at[p]], rows_v.at[p], gsems.at[p])

        row0 = wid * n_per_w

        def outer(j0, carry):
            for b in range(NB):
                j = j0 * NB + b

                pltpu.make_async_copy(
                    table_hbm.at[idx_v.at[j]], rows_v.at[b], gsems.at[b]
                ).wait()
                pltpu.async_copy(
                    rows_v.at[b],
                    out_hbm.at[pl.ds(row0 + j * CHUNK, CHUNK)],
                    osems.at[b],
                )

                nxt = (b - 1) % NB

                @pl.when(j + NB - 1 < n_chunks)
                def _():
                    @pl.when(j >= 1)
                    def _():
                        pltpu.make_async_copy(
                            rows_v.at[nxt],
                            out_hbm.at[pl.ds(0, CHUNK)],
                            osems.at[nxt],
                        ).wait()

                    pltpu.async_copy(
                        table_hbm.at[idx_v.at[j + NB - 1]],
                        rows_v.at[nxt],
                        gsems.at[nxt],
                    )
            return carry

        lax.fori_loop(0, n_chunks // NB, outer, 0)

        for b in range(NB):
            pltpu.make_async_copy(
                rows_v.at[b], out_hbm.at[pl.ds(0, CHUNK)], osems.at[b]
            ).wait()

    grid_kernel = pl.kernel(
        body,
        out_type=jax.ShapeDtypeStruct((B, D), jnp.float32),
        mesh=mesh,
        scratch_types=[
            pltpu.VMEM((n_chunks, CHUNK), jnp.int32),
            pltpu.VMEM((NB, CHUNK, D), jnp.float32),
            pltpu.SemaphoreType.DMA((NB,)),
            pltpu.SemaphoreType.DMA((NB,)),
        ],
    )
    return grid_kernel


def kernel(x, embedding):
    orig_shape = x.shape
    flat = x.reshape(-1)
    B = flat.shape[0]
    idx3d = flat.reshape(NW, B // (NW * CHUNK), CHUNK)
    out = _build(B)(embedding, idx3d)
    return out.reshape(*orig_shape, D)


# speedup vs baseline: 4.7061x; 1.6195x over previous
import jax
import jax.numpy as jnp
from jax import lax
from jax.experimental import pallas as pl
from jax.experimental.pallas import tpu as pltpu
from jax.experimental.pallas import tpu_sc as plsc

NC = 2
NS = 16
NW = NC * NS

D = 128
GROUP = 8


def _build(N, S):
    xpw = N // NW
    gpw = xpw // GROUP

    mesh = plsc.VectorSubcoreMesh(core_axis_name="c", subcore_axis_name="s")

    def body(x_hbm, table_hbm, out_hbm, idx_v, buf, gsems, osems):
        wid = lax.axis_index("s") * NC + lax.axis_index("c")
        xrow0 = wid * xpw
        pltpu.sync_copy(x_hbm.at[pl.ds(xrow0, xpw)], idx_v)
        for k in range(GROUP):
            pltpu.async_copy(table_hbm.at[idx_v.at[k]], buf.at[0, k], gsems.at[0])

        def step(g, carry):
            b = lax.rem(g, 2)
            nb = 1 - b

            @pl.when(g + 1 < gpw)
            def _():
                @pl.when(g >= 1)
                def _():
                    pltpu.make_async_copy(
                        buf.at[nb], out_hbm.at[pl.ds(0, GROUP)], osems.at[nb]
                    ).wait()

                for k in range(GROUP):
                    pltpu.async_copy(
                        table_hbm.at[idx_v.at[(g + 1) * GROUP + k]],
                        buf.at[nb, k],
                        gsems.at[nb],
                    )

            for k in range(GROUP):
                pltpu.make_async_copy(
                    table_hbm.at[idx_v.at[g * GROUP + k]], buf.at[b, k], gsems.at[b]
                ).wait()
            pltpu.async_copy(
                buf.at[b],
                out_hbm.at[pl.ds(xrow0 + g * GROUP, GROUP)],
                osems.at[b],
            )
            return carry

        lax.fori_loop(0, gpw, step, 0)
        pltpu.make_async_copy(
            buf.at[(gpw - 1) % 2], out_hbm.at[pl.ds(0, GROUP)], osems.at[(gpw - 1) % 2]
        ).wait()

    grid_kernel = pl.kernel(
        body,
        out_type=jax.ShapeDtypeStruct((N, S, D), jnp.float32),
        mesh=mesh,
        scratch_types=[
            pltpu.VMEM((xpw, S), jnp.int32),
            pltpu.VMEM((2, GROUP, S, D), jnp.float32),
            pltpu.SemaphoreType.DMA((2,)),
            pltpu.SemaphoreType.DMA((2,)),
        ],
    )
    return grid_kernel


def kernel(x, embedding):
    N, S = x.shape
    return _build(N, S)(x, embedding)


# speedup vs baseline: 7.2787x; 1.5467x over previous
import jax
import jax.numpy as jnp
from jax import lax
from jax.experimental import pallas as pl
from jax.experimental.pallas import tpu as pltpu
from jax.experimental.pallas import tpu_sc as plsc

NC = 2
NS = 16
NW = NC * NS

D = 128
NB = 4


def _build(S, N):
    npw = N // NW

    mesh = plsc.VectorSubcoreMesh(core_axis_name="c", subcore_axis_name="s")

    def body(xt_hbm, table_hbm, out_hbm, idx_v, buf, gsems, osems):
        wid = lax.axis_index("s") * NC + lax.axis_index("c")
        n0 = wid * npw
        pltpu.sync_copy(xt_hbm.at[:, pl.ds(n0, npw)], idx_v)
        for p in range(NB - 1):
            pltpu.async_copy(table_hbm.at[idx_v.at[p]], buf.at[p], gsems.at[p])

        def step(j, carry):
            b = lax.rem(j, NB)

            pltpu.make_async_copy(
                table_hbm.at[idx_v.at[j]], buf.at[b], gsems.at[b]
            ).wait()
            pltpu.async_copy(
                buf.at[b], out_hbm.at[j, pl.ds(n0, npw)], osems.at[b]
            )

            nb = lax.rem(b + NB - 1, NB)

            @pl.when(j + NB - 1 < S)
            def _():
                @pl.when(j >= 1)
                def _():
                    pltpu.make_async_copy(
                        buf.at[nb], out_hbm.at[0, pl.ds(0, npw)], osems.at[nb]
                    ).wait()

                pltpu.async_copy(
                    table_hbm.at[idx_v.at[j + NB - 1]], buf.at[nb], gsems.at[nb]
                )

            return carry

        lax.fori_loop(0, S, step, 0)
        for b in range(NB):
            pltpu.make_async_copy(
                buf.at[b], out_hbm.at[0, pl.ds(0, npw)], osems.at[b]
            ).wait()

    grid_kernel = pl.kernel(
        body,
        out_type=jax.ShapeDtypeStruct((S, N, D), jnp.float32),
        mesh=mesh,
        scratch_types=[
            pltpu.VMEM((S, npw), jnp.int32),
            pltpu.VMEM((NB, npw, D), jnp.float32),
            pltpu.SemaphoreType.DMA((NB,)),
            pltpu.SemaphoreType.DMA((NB,)),
        ],
    )
    return grid_kernel


def kernel(x, embedding):
    N, S = x.shape
    out_phys = _build(S, N)(x.T, embedding)
    return out_phys.transpose(1, 0, 2)


# speedup vs baseline: 15.8580x; 2.1787x over previous
import jax
import jax.numpy as jnp
from jax import lax
from jax.experimental import pallas as pl
from jax.experimental.pallas import tpu as pltpu
from jax.experimental.pallas import tpu_sc as plsc

NC = 2
NS = 16
NW = NC * NS

D = 128
NB = 4


def _build(S, N, V):
    npw = N // NW

    mesh = plsc.VectorSubcoreMesh(core_axis_name="c", subcore_axis_name="s")

    def body(xt_hbm, table_hbm, out_hbm, idx_v, buf, table_sh, gsems, osems):
        wid = lax.axis_index("s") * NC + lax.axis_index("c")
        n0 = wid * npw
        @pl.when(lax.axis_index("s") == 0)
        def _():
            pltpu.sync_copy(table_hbm, table_sh)

        pltpu.sync_copy(xt_hbm.at[:, pl.ds(n0, npw)], idx_v)
        plsc.subcore_barrier()
        for p in range(NB - 1):
            pltpu.async_copy(table_sh.at[idx_v.at[p]], buf.at[p], gsems.at[p])

        def step(j, carry):
            b = lax.rem(j, NB)

            pltpu.make_async_copy(
                table_sh.at[idx_v.at[j]], buf.at[b], gsems.at[b]
            ).wait()
            pltpu.async_copy(
                buf.at[b], out_hbm.at[j, pl.ds(n0, npw)], osems.at[b]
            )

            nb = lax.rem(b + NB - 1, NB)

            @pl.when(j + NB - 1 < S)
            def _():
                @pl.when(j >= 1)
                def _():
                    pltpu.make_async_copy(
                        buf.at[nb], out_hbm.at[0, pl.ds(0, npw)], osems.at[nb]
                    ).wait()

                pltpu.async_copy(
                    table_sh.at[idx_v.at[j + NB - 1]], buf.at[nb], gsems.at[nb]
                )

            return carry

        lax.fori_loop(0, S, step, 0)
        for b in range(NB):
            pltpu.make_async_copy(
                buf.at[b], out_hbm.at[0, pl.ds(0, npw)], osems.at[b]
            ).wait()

    grid_kernel = pl.kernel(
        body,
        out_type=jax.ShapeDtypeStruct((S, N, D), jnp.float32),
        mesh=mesh,
        scratch_types=[
            pltpu.VMEM((S, npw), jnp.int32),
            pltpu.VMEM((NB, npw, D), jnp.float32),
            pltpu.VMEM_SHARED((V, D), jnp.float32),
            pltpu.SemaphoreType.DMA((NB,)),
            pltpu.SemaphoreType.DMA((NB,)),
        ],
    )
    return grid_kernel


def kernel(x, embedding):
    N, S = x.shape
    out_phys = _build(S, N, embedding.shape[0])(x.T, embedding)
    return out_phys.transpose(1, 0, 2)
